# baseline (speedup 1.0000x reference)
import functools

import jax
import jax.numpy as jnp
from jax import lax
from jax.experimental import pallas as pl
from jax.experimental.pallas import tpu as pltpu
from jax.experimental.pallas import tpu_sc as plsc

VOCAB = 100000
EMB_DIM = 64
BATCH = 4096
LANGNUM = 2

_info = plsc.get_sparse_core_info()
_NC, _NS, _NL = _info.num_cores, _info.num_subcores, _info.num_lanes
_TCW = 128
_NTC = (VOCAB + _TCW - 1) // _TCW
_KMAX = (_NTC + _NS - 1) // _NS
_LL = BATCH + 2 * _NL

_mesh = plsc.VectorSubcoreMesh(core_axis_name="c", subcore_axis_name="s")


@functools.partial(
    pl.kernel,
    mesh=_mesh,
    out_type=jax.ShapeDtypeStruct((LANGNUM, BATCH, 2 * EMB_DIM), jnp.float32),
    scratch_types=[
        pltpu.VMEM((BATCH,), jnp.int32),
        pltpu.VMEM((BATCH,), jnp.int32),
        pltpu.VMEM((4 * _LL,), jnp.int32),
        pltpu.VMEM((EMB_DIM, _TCW), jnp.float32),
        pltpu.VMEM((EMB_DIM, _TCW), jnp.float32),
        pltpu.VMEM((EMB_DIM, _TCW), jnp.float32),
        pltpu.VMEM((EMB_DIM, _TCW), jnp.float32),
        pltpu.VMEM((_NL, 2 * EMB_DIM), jnp.float32),
        pltpu.VMEM((_NL,), jnp.int32),
        pltpu.VMEM_SHARED((BATCH + _NL, 2 * EMB_DIM), jnp.float32),
        pltpu.SemaphoreType.DMA,
        pltpu.SemaphoreType.DMA,
        pltpu.SemaphoreType.DMA,
        pltpu.SemaphoreType.DMA,
        pltpu.SemaphoreType.DMA,
    ],
    compiler_params=pltpu.CompilerParams(
        use_tc_tiling_on_sc=True,
        disable_bounds_checks=True,
        disable_semaphore_checks=True,
        needs_layout_passes=False,
    ),
)
def _embed_sc(idx0_hbm, idx1_hbm, emb0t_hbm, emb1t_hbm, out_hbm,
              idx0_v, idx1_v, lists_v, chunk_a, chunk_b, chunk_c, chunk_d,
              rowbuf_v, sidx_v, smem_out, sem, sem_a, sem_b, sem_c, sem_d):
    lang = lax.axis_index("c")
    s = lax.axis_index("s")
    lanes = lax.iota(jnp.int32, _NL)
    trash = BATCH + _NL
    bidx_v = lists_v.at[pl.ds(0, _LL)]
    bpos_v = lists_v.at[pl.ds(_LL, _LL)]
    aoff_v = lists_v.at[pl.ds(2 * _LL, _LL)]
    apos_v = lists_v.at[pl.ds(3 * _LL, _LL)]

    pltpu.sync_copy(idx0_hbm, idx0_v)
    pltpu.sync_copy(idx1_hbm, idx1_v)

    def do_lang(idx_v, tab_hbm):
        def bucket(v, off):
            vec = idx_v[pl.ds(v * _NL, _NL)]
            tc = lax.shift_right_logical(vec, 7)
            mask = (tc & (_NS - 1)) == s
            pos = lanes + v * _NL
            cm = plsc.cumsum(mask.astype(jnp.int32))
            dst = jnp.where(mask, off + cm - 1, trash)
            plsc.store_scatter(bidx_v, [dst], vec)
            plsc.store_scatter(bpos_v, [dst], pos)
            return off + cm[_NL - 1]

        nb = lax.fori_loop(0, BATCH // _NL, bucket, 0)
        bidx_v[pl.ds(nb, _NL)] = jnp.full((_NL,), -1, jnp.int32)

        def start_fetch(k, chunk_v, csem):
            tc = s + k * _NS

            @pl.when(tc < _NTC)
            def _():
                start = pl.multiple_of(tc * _TCW, _TCW)
                pltpu.make_async_copy(
                    tab_hbm.at[:, pl.ds(start, _TCW)], chunk_v, csem).start()

        def process(k, chunk_v, csem):
            tc = s + k * _NS

            @pl.when(tc < _NTC)
            def _():
                pltpu.make_async_copy(
                    tab_hbm.at[:, pl.ds(0, _TCW)], chunk_v, csem).wait()

                def scan(g, na):
                    vec = bidx_v[pl.ds(g * _NL, _NL)]
                    m = lax.shift_right_logical(vec, 7) == tc
                    cm = plsc.cumsum(m.astype(jnp.int32))
                    dst = jnp.where(m, na + cm - 1, trash)
                    plsc.store_scatter(aoff_v, [dst], vec & (_TCW - 1))
                    plsc.store_scatter(
                        apos_v, [dst], bpos_v[pl.ds(g * _NL, _NL)])
                    return na + cm[_NL - 1]

                na = lax.fori_loop(0, (nb + _NL - 1) // _NL, scan, 0)
                aoff_v[pl.ds(na, _NL)] = jnp.zeros((_NL,), jnp.int32)
                apos_v[pl.ds(na, _NL)] = lanes + BATCH

                def group(g, _):
                    offv = aoff_v[pl.ds(g * _NL, _NL)]
                    sidx_v[...] = apos_v[pl.ds(g * _NL, _NL)]
                    for t in range(_NL):
                        col = jnp.broadcast_to(offv[t], (_NL,))
                        for fg in range(EMB_DIM // _NL):
                            rows = lanes + fg * _NL
                            vals = plsc.load_gather(chunk_v, [rows, col])
                            plsc.store_scatter(
                                rowbuf_v,
                                [jnp.broadcast_to(jnp.int32(t), (_NL,)),
                                 rows], vals)
                    pltpu.make_async_copy(
                        rowbuf_v, smem_out.at[sidx_v], sem).start()
                    pltpu.make_async_copy(
                        rowbuf_v, smem_out.at[sidx_v], sem).wait()
                    return 0

                lax.fori_loop(0, (na + _NL - 1) // _NL, group, 0)

        bufs = ((chunk_a, sem_a), (chunk_b, sem_b),
                (chunk_c, sem_c), (chunk_d, sem_d))
        for j, (cv, cs) in enumerate(bufs):
            start_fetch(j, cv, cs)

        def pipe(i, _):
            k0 = 4 * i
            for j, (cv, cs) in enumerate(bufs):
                process(k0 + j, cv, cs)
                start_fetch(k0 + j + 4, cv, cs)
            return 0

        lax.fori_loop(0, (_KMAX + 3) // 4, pipe, 0)

    @pl.when(lang == 0)
    def _():
        do_lang(idx0_v, emb0t_hbm)

    @pl.when(lang == 1)
    def _():
        do_lang(idx1_v, emb1t_hbm)

    plsc.subcore_barrier()
    stripe = BATCH // _NS

    @pl.when(lang == 0)
    def _():
        pltpu.sync_copy(
            smem_out.at[pl.ds(s * stripe, stripe)],
            out_hbm.at[0, pl.ds(s * stripe, stripe)])

    @pl.when(lang == 1)
    def _():
        pltpu.sync_copy(
            smem_out.at[pl.ds(s * stripe, stripe)],
            out_hbm.at[1, pl.ds(s * stripe, stripe)])


def kernel(idx0, idx1, emb0, emb1):
    out = _embed_sc(idx0, idx1, emb0.T, emb1.T)
    return out[:, :, :EMB_DIM]


# speedup vs baseline: 1.4614x; 1.4614x over previous
import functools

import jax
import jax.numpy as jnp
from jax import lax
from jax.experimental import pallas as pl
from jax.experimental.pallas import tpu as pltpu
from jax.experimental.pallas import tpu_sc as plsc

VOCAB = 100000
EMB_DIM = 64
BATCH = 4096
LANGNUM = 2

_info = plsc.get_sparse_core_info()
_NC, _NS, _NL = _info.num_cores, _info.num_subcores, _info.num_lanes
_TCW = 128
_NTC = (VOCAB + _TCW - 1) // _TCW
_KMAX = (_NTC + _NS - 1) // _NS
_SCW = 4
_NSC = (_KMAX + _SCW - 1) // _SCW
_RM = 21400
_LL = BATCH + 2 * _NL

_mesh = plsc.VectorSubcoreMesh(core_axis_name="c", subcore_axis_name="s")


@functools.partial(
    pl.kernel,
    mesh=_mesh,
    out_type=jax.ShapeDtypeStruct((LANGNUM, BATCH, 2 * EMB_DIM), jnp.float32),
    scratch_types=[
        pltpu.VMEM((BATCH,), jnp.int32),
        pltpu.VMEM((BATCH,), jnp.int32),
        pltpu.VMEM((4 * _LL,), jnp.int32),
        pltpu.VMEM((EMB_DIM, _SCW * _TCW), jnp.float32),
        pltpu.VMEM((EMB_DIM, _SCW * _TCW), jnp.float32),
        pltpu.VMEM((_NL, 2 * EMB_DIM), jnp.float32),
        pltpu.VMEM((_NL,), jnp.int32),
        pltpu.VMEM_SHARED((BATCH + _NL, 2 * EMB_DIM), jnp.float32),
        pltpu.SemaphoreType.DMA,
        pltpu.SemaphoreType.DMA,
        pltpu.SemaphoreType.DMA,
    ],
    compiler_params=pltpu.CompilerParams(
        use_tc_tiling_on_sc=True,
        disable_bounds_checks=True,
        disable_semaphore_checks=True,
        needs_layout_passes=False,
    ),
)
def _embed_sc(idx0_hbm, idx1_hbm, emb0t_hbm, emb1t_hbm, out_hbm,
              idx0_v, idx1_v, lists_v, chunk_a, chunk_b,
              rowbuf_v, sidx_v, smem_out, sem, sem_a, sem_b):
    lang = lax.axis_index("c")
    s = lax.axis_index("s")
    lanes = lax.iota(jnp.int32, _NL)
    trash = BATCH + _NL
    bidx_v = lists_v.at[pl.ds(0, _LL)]
    bpos_v = lists_v.at[pl.ds(_LL, _LL)]
    aoff_v = lists_v.at[pl.ds(2 * _LL, _LL)]
    apos_v = lists_v.at[pl.ds(3 * _LL, _LL)]

    pltpu.sync_copy(idx0_hbm, idx0_v)
    pltpu.sync_copy(idx1_hbm, idx1_v)

    def do_lang(idx_v, tab_hbm):
        def bucket(v, off):
            vec = idx_v[pl.ds(v * _NL, _NL)]
            tc = lax.shift_right_logical(vec, 7)
            owner = lax.shift_right_logical(tc * _RM, 20)
            mask = owner == s
            pos = lanes + v * _NL
            cm = plsc.cumsum(mask.astype(jnp.int32))
            dst = jnp.where(mask, off + cm - 1, trash)
            plsc.store_scatter(bidx_v, [dst], vec)
            plsc.store_scatter(bpos_v, [dst], pos)
            return off + cm[_NL - 1]

        nb = lax.fori_loop(0, BATCH // _NL, bucket, 0)
        bidx_v[pl.ds(nb, _NL)] = jnp.full((_NL,), -1, jnp.int32)

        tend = jnp.minimum((s + 1) * _KMAX, _NTC)

        def start_fetch(m, chunk_v, csem):
            tc0 = s * _KMAX + m * _SCW

            @pl.when((tc0 < tend) & (tc0 + _SCW <= _NTC))
            def _():
                start = pl.multiple_of(tc0 * _TCW, _TCW)
                pltpu.make_async_copy(
                    tab_hbm.at[:, pl.ds(start, _SCW * _TCW)],
                    chunk_v, csem).start()

            @pl.when((tc0 < tend) & (tc0 + _SCW > _NTC))
            def _():
                for j in range(_SCW - 1):
                    @pl.when(tc0 + j < _NTC)
                    def _():
                        start = pl.multiple_of((tc0 + j) * _TCW, _TCW)
                        pltpu.make_async_copy(
                            tab_hbm.at[:, pl.ds(start, _TCW)],
                            chunk_v.at[:, pl.ds(j * _TCW, _TCW)],
                            csem).start()

        def process(m, chunk_v, csem):
            tc0 = s * _KMAX + m * _SCW

            @pl.when((tc0 < tend) & (tc0 + _SCW <= _NTC))
            def _():
                pltpu.make_async_copy(
                    tab_hbm.at[:, pl.ds(0, _SCW * _TCW)],
                    chunk_v, csem).wait()

            @pl.when((tc0 < tend) & (tc0 + _SCW > _NTC))
            def _():
                for j in range(_SCW - 1):
                    @pl.when(tc0 + j < _NTC)
                    def _():
                        pltpu.make_async_copy(
                            tab_hbm.at[:, pl.ds(0, _TCW)],
                            chunk_v.at[:, pl.ds(j * _TCW, _TCW)],
                            csem).wait()

            @pl.when(tc0 < tend)
            def _():
                base_col = tc0 * _TCW

                def scan(g, na):
                    vec = bidx_v[pl.ds(g * _NL, _NL)]
                    off = vec - base_col
                    m2 = (vec >= base_col) & (off < _SCW * _TCW)
                    cm = plsc.cumsum(m2.astype(jnp.int32))
                    dst = jnp.where(m2, na + cm - 1, trash)
                    plsc.store_scatter(aoff_v, [dst], off)
                    plsc.store_scatter(
                        apos_v, [dst], bpos_v[pl.ds(g * _NL, _NL)])
                    return na + cm[_NL - 1]

                na = lax.fori_loop(0, (nb + _NL - 1) // _NL, scan, 0)
                aoff_v[pl.ds(na, _NL)] = jnp.zeros((_NL,), jnp.int32)
                apos_v[pl.ds(na, _NL)] = lanes + BATCH

                def group(g, _):
                    offv = aoff_v[pl.ds(g * _NL, _NL)]
                    sidx_v[...] = apos_v[pl.ds(g * _NL, _NL)]
                    for t in range(_NL):
                        col = jnp.broadcast_to(offv[t], (_NL,))
                        for fg in range(EMB_DIM // _NL):
                            rows = lanes + fg * _NL
                            vals = plsc.load_gather(chunk_v, [rows, col])
                            plsc.store_scatter(
                                rowbuf_v,
                                [jnp.broadcast_to(jnp.int32(t), (_NL,)),
                                 rows], vals)
                    pltpu.make_async_copy(
                        rowbuf_v, smem_out.at[sidx_v], sem).start()
                    pltpu.make_async_copy(
                        rowbuf_v, smem_out.at[sidx_v], sem).wait()
                    return 0

                lax.fori_loop(0, (na + _NL - 1) // _NL, group, 0)

        start_fetch(0, chunk_a, sem_a)

        def pipe(i, _):
            m0 = 2 * i
            start_fetch(m0 + 1, chunk_b, sem_b)
            process(m0, chunk_a, sem_a)
            start_fetch(m0 + 2, chunk_a, sem_a)
            process(m0 + 1, chunk_b, sem_b)
            return 0

        lax.fori_loop(0, (_NSC + 1) // 2, pipe, 0)

    @pl.when(lang == 0)
    def _():
        do_lang(idx0_v, emb0t_hbm)

    @pl.when(lang == 1)
    def _():
        do_lang(idx1_v, emb1t_hbm)

    plsc.subcore_barrier()
    stripe = BATCH // _NS

    @pl.when(lang == 0)
    def _():
        pltpu.sync_copy(
            smem_out.at[pl.ds(s * stripe, stripe)],
            out_hbm.at[0, pl.ds(s * stripe, stripe)])

    @pl.when(lang == 1)
    def _():
        pltpu.sync_copy(
            smem_out.at[pl.ds(s * stripe, stripe)],
            out_hbm.at[1, pl.ds(s * stripe, stripe)])


def kernel(idx0, idx1, emb0, emb1):
    out = _embed_sc(idx0, idx1, emb0.T, emb1.T)
    return out[:, :, :EMB_DIM]


# speedup vs baseline: 1.4722x; 1.0074x over previous
import functools

import jax
import jax.numpy as jnp
from jax import lax
from jax.experimental import pallas as pl
from jax.experimental.pallas import tpu as pltpu
from jax.experimental.pallas import tpu_sc as plsc

VOCAB = 100000
EMB_DIM = 64
BATCH = 4096
LANGNUM = 2

_info = plsc.get_sparse_core_info()
_NC, _NS, _NL = _info.num_cores, _info.num_subcores, _info.num_lanes
_TCW = 128
_NTC = (VOCAB + _TCW - 1) // _TCW
_KMAX = (_NTC + _NS - 1) // _NS
_SCW = 4
_NSC = (_KMAX + _SCW - 1) // _SCW
_RM = 21400
_LL = BATCH + 2 * _NL

_mesh = plsc.VectorSubcoreMesh(core_axis_name="c", subcore_axis_name="s")


@functools.partial(
    pl.kernel,
    mesh=_mesh,
    out_type=jax.ShapeDtypeStruct((LANGNUM, BATCH, 2 * EMB_DIM), jnp.float32),
    scratch_types=[
        pltpu.VMEM((BATCH,), jnp.int32),
        pltpu.VMEM((BATCH,), jnp.int32),
        pltpu.VMEM((4 * _LL,), jnp.int32),
        pltpu.VMEM((EMB_DIM, _SCW * _TCW), jnp.float32),
        pltpu.VMEM((EMB_DIM, _SCW * _TCW), jnp.float32),
        pltpu.VMEM((_NL, 2 * EMB_DIM), jnp.float32),
        pltpu.VMEM((_NL,), jnp.int32),
        pltpu.VMEM_SHARED((BATCH + _NL, 2 * EMB_DIM), jnp.float32),
        pltpu.SemaphoreType.DMA,
        pltpu.SemaphoreType.DMA,
        pltpu.SemaphoreType.DMA,
    ],
    compiler_params=pltpu.CompilerParams(
        use_tc_tiling_on_sc=True,
        disable_bounds_checks=True,
        disable_semaphore_checks=True,
        needs_layout_passes=False,
    ),
)
def _embed_sc(idx0_hbm, idx1_hbm, emb0t_hbm, emb1t_hbm, out_hbm,
              idx0_v, idx1_v, lists_v, chunk_a, chunk_b,
              rowbuf_v, sidx_v, smem_out, sem, sem_a, sem_b):
    lang = lax.axis_index("c")
    s = lax.axis_index("s")
    lanes = lax.iota(jnp.int32, _NL)
    trash = BATCH + _NL
    bidx_v = lists_v.at[pl.ds(0, _LL)]
    bpos_v = lists_v.at[pl.ds(_LL, _LL)]
    aoff_v = lists_v.at[pl.ds(2 * _LL, _LL)]
    apos_v = lists_v.at[pl.ds(3 * _LL, _LL)]

    pltpu.sync_copy(idx0_hbm, idx0_v)
    pltpu.sync_copy(idx1_hbm, idx1_v)

    def do_lang(idx_v, tab_hbm):
        def bucket(v, off):
            vec = idx_v[pl.ds(v * _NL, _NL)]
            tc = lax.shift_right_logical(vec, 7)
            owner = lax.shift_right_logical(tc * _RM, 20)
            mask = owner == s
            pos = lanes + v * _NL
            cm = plsc.cumsum(mask.astype(jnp.int32))
            dst = jnp.where(mask, off + cm - 1, trash)
            plsc.store_scatter(bidx_v, [dst], vec)
            plsc.store_scatter(bpos_v, [dst], pos)
            return off + cm[_NL - 1]

        nb = lax.fori_loop(0, BATCH // _NL, bucket, 0)
        bidx_v[pl.ds(nb, _NL)] = jnp.full((_NL,), -1, jnp.int32)

        tend = jnp.minimum((s + 1) * _KMAX, _NTC)

        def start_fetch(m, chunk_v, csem):
            tc0 = s * _KMAX + m * _SCW

            @pl.when((tc0 < tend) & (tc0 + _SCW <= _NTC))
            def _():
                start = pl.multiple_of(tc0 * _TCW, _TCW)
                pltpu.make_async_copy(
                    tab_hbm.at[:, pl.ds(start, _SCW * _TCW)],
                    chunk_v, csem).start()

            @pl.when((tc0 < tend) & (tc0 + _SCW > _NTC))
            def _():
                for j in range(_SCW - 1):
                    @pl.when(tc0 + j < _NTC)
                    def _():
                        start = pl.multiple_of((tc0 + j) * _TCW, _TCW)
                        pltpu.make_async_copy(
                            tab_hbm.at[:, pl.ds(start, _TCW)],
                            chunk_v.at[:, pl.ds(j * _TCW, _TCW)],
                            csem).start()

        def process(m, chunk_v, csem):
            tc0 = s * _KMAX + m * _SCW

            @pl.when((tc0 < tend) & (tc0 + _SCW <= _NTC))
            def _():
                pltpu.make_async_copy(
                    tab_hbm.at[:, pl.ds(0, _SCW * _TCW)],
                    chunk_v, csem).wait()

            @pl.when((tc0 < tend) & (tc0 + _SCW > _NTC))
            def _():
                for j in range(_SCW - 1):
                    @pl.when(tc0 + j < _NTC)
                    def _():
                        pltpu.make_async_copy(
                            tab_hbm.at[:, pl.ds(0, _TCW)],
                            chunk_v.at[:, pl.ds(j * _TCW, _TCW)],
                            csem).wait()

            @pl.when(tc0 < tend)
            def _():
                base_col = tc0 * _TCW

                def scan(g, na):
                    vec = bidx_v[pl.ds(g * _NL, _NL)]
                    off = vec - base_col
                    m2 = (vec >= base_col) & (off < _SCW * _TCW)
                    cm = plsc.cumsum(m2.astype(jnp.int32))
                    dst = jnp.where(m2, na + cm - 1, trash)
                    plsc.store_scatter(aoff_v, [dst], off)
                    plsc.store_scatter(
                        apos_v, [dst], bpos_v[pl.ds(g * _NL, _NL)])
                    return na + cm[_NL - 1]

                na = lax.fori_loop(0, (nb + _NL - 1) // _NL, scan, 0)
                aoff_v[pl.ds(na, _NL)] = jnp.zeros((_NL,), jnp.int32)
                apos_v[pl.ds(na, _NL)] = lanes + BATCH

                def group(g, _):
                    offv = aoff_v[pl.ds(g * _NL, _NL)]
                    sidx_v[...] = apos_v[pl.ds(g * _NL, _NL)]
                    for f in range(EMB_DIM):
                        frow = jnp.full((_NL,), f, jnp.int32)
                        vals = plsc.load_gather(chunk_v, [frow, offv])
                        plsc.store_scatter(rowbuf_v, [lanes, frow], vals)
                    pltpu.make_async_copy(
                        rowbuf_v, smem_out.at[sidx_v], sem).start()
                    pltpu.make_async_copy(
                        rowbuf_v, smem_out.at[sidx_v], sem).wait()
                    return 0

                lax.fori_loop(0, (na + _NL - 1) // _NL, group, 0)

        start_fetch(0, chunk_a, sem_a)

        def pipe(i, _):
            m0 = 2 * i
            start_fetch(m0 + 1, chunk_b, sem_b)
            process(m0, chunk_a, sem_a)
            start_fetch(m0 + 2, chunk_a, sem_a)
            process(m0 + 1, chunk_b, sem_b)
            return 0

        lax.fori_loop(0, (_NSC + 1) // 2, pipe, 0)

    @pl.when(lang == 0)
    def _():
        do_lang(idx0_v, emb0t_hbm)

    @pl.when(lang == 1)
    def _():
        do_lang(idx1_v, emb1t_hbm)

    plsc.subcore_barrier()
    stripe = BATCH // _NS

    @pl.when(lang == 0)
    def _():
        pltpu.sync_copy(
            smem_out.at[pl.ds(s * stripe, stripe)],
            out_hbm.at[0, pl.ds(s * stripe, stripe)])

    @pl.when(lang == 1)
    def _():
        pltpu.sync_copy(
            smem_out.at[pl.ds(s * stripe, stripe)],
            out_hbm.at[1, pl.ds(s * stripe, stripe)])


def kernel(idx0, idx1, emb0, emb1):
    out = _embed_sc(idx0, idx1, emb0.T, emb1.T)
    return out[:, :, :EMB_DIM]


# speedup vs baseline: 1.5060x; 1.0230x over previous
import functools

import jax
import jax.numpy as jnp
from jax import lax
from jax.experimental import pallas as pl
from jax.experimental.pallas import tpu as pltpu
from jax.experimental.pallas import tpu_sc as plsc

VOCAB = 100000
EMB_DIM = 64
BATCH = 4096
LANGNUM = 2

_info = plsc.get_sparse_core_info()
_NC, _NS, _NL = _info.num_cores, _info.num_subcores, _info.num_lanes
_TCW = 128
_NTC = (VOCAB + _TCW - 1) // _TCW
_KMAX = (_NTC + _NS - 1) // _NS
_SCW = 4
_NSC = (_KMAX + _SCW - 1) // _SCW
_RM = 21400
_LL = BATCH + 2 * _NL

_mesh = plsc.VectorSubcoreMesh(core_axis_name="c", subcore_axis_name="s")


@functools.partial(
    pl.kernel,
    mesh=_mesh,
    out_type=jax.ShapeDtypeStruct((LANGNUM, BATCH, 2 * EMB_DIM), jnp.float32),
    scratch_types=[
        pltpu.VMEM((BATCH,), jnp.int32),
        pltpu.VMEM((4 * _LL,), jnp.int32),
        pltpu.VMEM((EMB_DIM, _SCW * _TCW), jnp.float32),
        pltpu.VMEM((EMB_DIM, _SCW * _TCW), jnp.float32),
        pltpu.VMEM((_NL, 2 * EMB_DIM), jnp.float32),
        pltpu.VMEM((_NL,), jnp.int32),
        pltpu.VMEM_SHARED((BATCH + _NL, 2 * EMB_DIM), jnp.float32),
        pltpu.SemaphoreType.DMA,
        pltpu.SemaphoreType.DMA,
        pltpu.SemaphoreType.DMA,
    ],
    compiler_params=pltpu.CompilerParams(
        use_tc_tiling_on_sc=True,
        disable_bounds_checks=True,
        disable_semaphore_checks=True,
        needs_layout_passes=False,
    ),
)
def _embed_sc(idx0_hbm, idx1_hbm, emb0t_hbm, emb1t_hbm, out_hbm,
              idx_v, lists_v, chunk_a, chunk_b,
              rowbuf_v, sidx_v, smem_out, sem, sem_a, sem_b):
    lang = lax.axis_index("c")
    s = lax.axis_index("s")
    lanes = lax.iota(jnp.int32, _NL)
    trash = BATCH + _NL
    bidx_v = lists_v.at[pl.ds(0, _LL)]
    bpos_v = lists_v.at[pl.ds(_LL, _LL)]
    aoff_v = lists_v.at[pl.ds(2 * _LL, _LL)]
    apos_v = lists_v.at[pl.ds(3 * _LL, _LL)]

    def do_lang(idx_hbm, tab_hbm):
        pltpu.sync_copy(idx_hbm, idx_v)
        def bucket(i, off):
            vec0 = idx_v[pl.ds((2 * i) * _NL, _NL)]
            vec1 = idx_v[pl.ds((2 * i + 1) * _NL, _NL)]
            m0 = lax.shift_right_logical(
                lax.shift_right_logical(vec0, 7) * _RM, 20) == s
            m1 = lax.shift_right_logical(
                lax.shift_right_logical(vec1, 7) * _RM, 20) == s
            cm0 = plsc.cumsum(m0.astype(jnp.int32))
            cm1 = plsc.cumsum(m1.astype(jnp.int32))
            dst0 = jnp.where(m0, off + cm0 - 1, trash)
            plsc.store_scatter(bidx_v, [dst0], vec0)
            plsc.store_scatter(bpos_v, [dst0], lanes + (2 * i) * _NL)
            off1 = off + cm0[_NL - 1]
            dst1 = jnp.where(m1, off1 + cm1 - 1, trash)
            plsc.store_scatter(bidx_v, [dst1], vec1)
            plsc.store_scatter(bpos_v, [dst1], lanes + (2 * i + 1) * _NL)
            return off1 + cm1[_NL - 1]

        nb = lax.fori_loop(0, BATCH // (2 * _NL), bucket, 0)
        bidx_v[pl.ds(nb, _NL)] = jnp.full((_NL,), -1, jnp.int32)

        tend = jnp.minimum((s + 1) * _KMAX, _NTC)

        def start_fetch(m, chunk_v, csem):
            tc0 = s * _KMAX + m * _SCW

            @pl.when((tc0 < tend) & (tc0 + _SCW <= _NTC))
            def _():
                start = pl.multiple_of(tc0 * _TCW, _TCW)
                pltpu.make_async_copy(
                    tab_hbm.at[:, pl.ds(start, _SCW * _TCW)],
                    chunk_v, csem).start()

            @pl.when((tc0 < tend) & (tc0 + _SCW > _NTC))
            def _():
                for j in range(_SCW - 1):
                    @pl.when(tc0 + j < _NTC)
                    def _():
                        start = pl.multiple_of((tc0 + j) * _TCW, _TCW)
                        pltpu.make_async_copy(
                            tab_hbm.at[:, pl.ds(start, _TCW)],
                            chunk_v.at[:, pl.ds(j * _TCW, _TCW)],
                            csem).start()

        def process(m, chunk_v, csem):
            tc0 = s * _KMAX + m * _SCW

            @pl.when((tc0 < tend) & (tc0 + _SCW <= _NTC))
            def _():
                pltpu.make_async_copy(
                    tab_hbm.at[:, pl.ds(0, _SCW * _TCW)],
                    chunk_v, csem).wait()

            @pl.when((tc0 < tend) & (tc0 + _SCW > _NTC))
            def _():
                for j in range(_SCW - 1):
                    @pl.when(tc0 + j < _NTC)
                    def _():
                        pltpu.make_async_copy(
                            tab_hbm.at[:, pl.ds(0, _TCW)],
                            chunk_v.at[:, pl.ds(j * _TCW, _TCW)],
                            csem).wait()

            @pl.when(tc0 < tend)
            def _():
                base_col = tc0 * _TCW

                def scan(g, na):
                    vec = bidx_v[pl.ds(g * _NL, _NL)]
                    off = vec - base_col
                    m2 = (vec >= base_col) & (off < _SCW * _TCW)
                    cm = plsc.cumsum(m2.astype(jnp.int32))
                    dst = jnp.where(m2, na + cm - 1, trash)
                    plsc.store_scatter(aoff_v, [dst], off)
                    plsc.store_scatter(
                        apos_v, [dst], bpos_v[pl.ds(g * _NL, _NL)])
                    return na + cm[_NL - 1]

                na = lax.fori_loop(0, (nb + _NL - 1) // _NL, scan, 0)
                aoff_v[pl.ds(na, _NL)] = jnp.zeros((_NL,), jnp.int32)
                apos_v[pl.ds(na, _NL)] = lanes + BATCH

                def group(g, _):
                    offv = aoff_v[pl.ds(g * _NL, _NL)]
                    sidx_v[...] = apos_v[pl.ds(g * _NL, _NL)]
                    for f in range(EMB_DIM):
                        frow = jnp.full((_NL,), f, jnp.int32)
                        vals = plsc.load_gather(chunk_v, [frow, offv])
                        plsc.store_scatter(rowbuf_v, [lanes, frow], vals)
                    pltpu.make_async_copy(
                        rowbuf_v, smem_out.at[sidx_v], sem).start()
                    pltpu.make_async_copy(
                        rowbuf_v, smem_out.at[sidx_v], sem).wait()
                    return 0

                lax.fori_loop(0, (na + _NL - 1) // _NL, group, 0)

        start_fetch(0, chunk_a, sem_a)

        def pipe(i, _):
            m0 = 2 * i
            start_fetch(m0 + 1, chunk_b, sem_b)
            process(m0, chunk_a, sem_a)
            start_fetch(m0 + 2, chunk_a, sem_a)
            process(m0 + 1, chunk_b, sem_b)
            return 0

        lax.fori_loop(0, (_NSC + 1) // 2, pipe, 0)

    @pl.when(lang == 0)
    def _():
        do_lang(idx0_hbm, emb0t_hbm)

    @pl.when(lang == 1)
    def _():
        do_lang(idx1_hbm, emb1t_hbm)

    plsc.subcore_barrier()
    stripe = BATCH // _NS

    @pl.when(lang == 0)
    def _():
        pltpu.sync_copy(
            smem_out.at[pl.ds(s * stripe, stripe)],
            out_hbm.at[0, pl.ds(s * stripe, stripe)])

    @pl.when(lang == 1)
    def _():
        pltpu.sync_copy(
            smem_out.at[pl.ds(s * stripe, stripe)],
            out_hbm.at[1, pl.ds(s * stripe, stripe)])


def kernel(idx0, idx1, emb0, emb1):
    out = _embed_sc(idx0, idx1, emb0.T, emb1.T)
    return out[:, :, :EMB_DIM]
